# baseline (speedup 1.0000x reference)
import functools

import jax
import jax.numpy as jnp
from jax import lax
from jax.experimental import pallas as pl
from jax.experimental.pallas import tpu as pltpu
from jax.experimental.pallas import tpu_sc as plsc

_CONF_TH = 0.85
_IGNORE = 2
_EPS = 1e-07
_DROP = 0.5

_ROWS = 2048
_COLS = 128
_N = _ROWS * _COLS

_LO1 = 0x3F400000
_SENTINEL_T = 0x41000000
_RANGE = _SENTINEL_T - _LO1

_NIMG = 8
_PARTS = 4
_CHUNK = _N // _PARTS

_SH_A = 14
_NB_A = _RANGE >> _SH_A
_SH_B = 3
_NB_B = 1 << (_SH_A - _SH_B)
_NB_C = 16

_L = 16
_HROW = 2048
_WIN = 8192


def _keys_from(p, mf):
    ann = mf != float(_IGNORE)
    conf = jnp.maximum(p, 1.0 - p)
    corr = (p > 0.5) == (mf == 1.0)
    isconf = conf > _CONF_TH
    score = jnp.ones_like(p)
    score = jnp.where(isconf & corr, 1.0, score)
    score = jnp.where((~isconf) & corr, 2.0, score)
    score = jnp.where((~isconf) & (~corr), 3.0, score)
    score = jnp.where(isconf & (~corr), 4.0, score)
    bonus = (conf - 0.5) * 0.5
    s = jnp.where(corr, score - bonus, score + bonus)
    key = jnp.where(ann, lax.bitcast_convert_type(s, jnp.int32), 0)
    return key, ann, conf, corr, isconf


def _pass_b(pred_ref, mask_ref, sel_ref, train_ref, hold_ref, part_ref):
    p = pred_ref[0]
    m = mask_ref[0]
    mf = m.astype(jnp.float32)
    key, ann, conf, corr, isconf = _keys_from(p, mf)

    t_key = sel_ref[0, 0, 0]
    m_star = sel_ref[0, 0, 1]

    rows = lax.broadcasted_iota(jnp.int32, (_ROWS, _COLS), 0)
    cols = lax.broadcasted_iota(jnp.int32, (_ROWS, _COLS), 1)
    fi = rows * _COLS + cols
    train = (key > t_key) | ((key == t_key) & (fi < m_star))
    hold = ann & (~train)

    pcl = jnp.clip(p, _EPS, 1.0 - _EPS)
    bce = -(mf * jnp.log(pcl) + (1.0 - mf) * jnp.log(1.0 - pcl))
    tf32 = train.astype(jnp.float32)
    hf32 = hold.astype(jnp.float32)

    cc = (isconf & corr).astype(jnp.float32)
    ci = (isconf & (~corr)).astype(jnp.float32)
    uc = ((~isconf) & corr).astype(jnp.float32)
    ui = ((~isconf) & (~corr)).astype(jnp.float32)

    den = jnp.sum(tf32)
    vals = [
        jnp.sum(bce * tf32),
        den,
        jnp.sum(cc * tf32),
        jnp.sum(ci * tf32),
        jnp.sum(uc * tf32),
        jnp.sum(ui * tf32),
        den,
        jnp.sum(cc * hf32),
        jnp.sum(ci * hf32),
        jnp.sum(uc * hf32),
        jnp.sum(ui * hf32),
        jnp.sum(hf32),
    ]
    col = lax.broadcasted_iota(jnp.int32, (1, _COLS), 1)
    out = jnp.zeros((1, _COLS), jnp.float32)
    for j, v in enumerate(vals):
        out = jnp.where(col == j, v, out)
    part_ref[0] = out

    train_ref[0] = train
    hold_ref[0] = hold


def _sc_select(pred_hbm, mask_hbm, sel_hbm, keys_v, hist_v, merged_v, winp, winm, shared):
    c = lax.axis_index("c")
    s = lax.axis_index("s")
    img = c * _PARTS + s // _PARTS
    part = s % _PARTS
    row = c * 16 + s
    g0 = (s // _PARTS) * _PARTS

    lanes = lax.iota(jnp.int32, _L)
    zero16 = jnp.zeros((_L,), jnp.int32)
    ones16 = jnp.ones((_L,), jnp.int32)

    def zero_hist(nb):
        @plsc.parallel_loop(0, nb, unroll=8)
        def _(i):
            hist_v[pl.ds(i * _L, _L)] = zero16

    def hist_pass(base, shift, nb, limit, low_only):
        off = lanes * nb

        @plsc.parallel_loop(0, _CHUNK // _L, unroll=8)
        def _(i):
            k = keys_v[pl.ds(i * _L, _L)]
            b = (k - base) >> shift
            if low_only:
                valid = b >= 0
            else:
                valid = (b >= 0) & (b < limit)
            plsc.addupdate_scatter(hist_v, [b + off], ones16, mask=valid)

    def collapse(nb):
        @plsc.parallel_loop(0, nb // _L, unroll=1)
        def _(j):
            acc = zero16
            for l in range(_L):
                acc = acc + hist_v[pl.ds(l * nb + j * _L, _L)]
            merged_v[pl.ds(j * _L, _L)] = acc

    def exchange_and_sum(nb, stage_off):
        plsc.subcore_barrier()
        pltpu.sync_copy(merged_v.at[pl.ds(0, nb)], shared.at[s, pl.ds(0, nb)])
        plsc.subcore_barrier()
        for t in range(_PARTS):
            pltpu.sync_copy(
                shared.at[g0 + t, pl.ds(0, nb)],
                hist_v.at[pl.ds(stage_off + t * _HROW, nb)],
            )

        @plsc.parallel_loop(0, nb // _L, unroll=2)
        def _(j):
            acc = zero16
            for t in range(_PARTS):
                acc = acc + hist_v[pl.ds(stage_off + t * _HROW + j * _L, _L)]
            merged_v[pl.ds(j * _L, _L)] = acc

    def suffix_scan(nb, need):
        def sbody(jj, carry):
            cry, bstar, rnext = carry
            j = nb // _L - 1 - jj
            v = merged_v[pl.ds(j * _L, _L)]
            q = lax.rev(plsc.cumsum(lax.rev(v, (0,))), (0,))
            suf = cry + q - v
            hit = (suf < need) & (suf + v >= need)
            bsel = jnp.max(jnp.where(hit, j * _L + lanes, -1))
            ssel = jnp.max(jnp.where(hit, suf, -1))
            bstar = jnp.where(bsel >= 0, bsel, bstar)
            rnext = jnp.where(bsel >= 0, need - ssel, rnext)
            return cry + jnp.sum(v), bstar, rnext

        _, bstar, rnext = lax.fori_loop(
            0, nb // _L, sbody, (jnp.int32(0), jnp.int32(-1), jnp.int32(-1))
        )
        return bstar, rnext

    zero_hist(_NB_A)
    off_a = lanes * _NB_A
    for w in range(_CHUNK // _WIN):
        pltpu.sync_copy(pred_hbm.at[row, pl.ds(w * _WIN, _WIN)], winp)
        pltpu.sync_copy(mask_hbm.at[row, pl.ds(w * _WIN, _WIN)], winm)

        @plsc.parallel_loop(0, _WIN // _L, unroll=8)
        def _(i):
            p = winp[pl.ds(i * _L, _L)]
            m = winm[pl.ds(i * _L, _L)]
            conf = jnp.maximum(p, 1.0 - p)
            corr = (p > 0.5) == (m == 1)
            isconf = conf > _CONF_TH
            score = jnp.where(
                corr,
                jnp.where(isconf, 1.0, 2.0),
                jnp.where(isconf, 4.0, 3.0),
            )
            bonus = (conf - 0.5) * 0.5
            sv = jnp.where(corr, score - bonus, score + bonus)
            key = jnp.where(
                m != _IGNORE, lax.bitcast_convert_type(sv, jnp.int32), 0
            )
            keys_v[pl.ds(w * _WIN + i * _L, _L)] = key
            b = (key - _LO1) >> _SH_A
            plsc.addupdate_scatter(hist_v, [b + off_a], ones16, mask=b >= 0)

    collapse(_NB_A)
    exchange_and_sum(_NB_A, 0)

    def abody(j, acc):
        return acc + merged_v[pl.ds(j * _L, _L)]

    a_tot = jnp.sum(lax.fori_loop(0, _NB_A // _L, abody, zero16))
    k_train = a_tot >> 1

    b_a, r_a = suffix_scan(_NB_A, k_train)
    base_b = _LO1 + b_a * (1 << _SH_A)

    zero_hist(_NB_B)
    hist_pass(base_b, _SH_B, _NB_B, _NB_B, False)
    collapse(_NB_B)
    exchange_and_sum(_NB_B, 0)
    b_b, r_b = suffix_scan(_NB_B, r_a)
    base_c = base_b + b_b * (1 << _SH_B)

    zero_hist(_NB_C)
    hist_pass(base_c, 0, _NB_C, 1 << _SH_B, False)
    collapse(_NB_C)
    exchange_and_sum(_NB_C, 8192)
    b_c, r_c = suffix_scan(_NB_C, r_b)
    t_key = base_c + b_c

    e_parts = []
    for t in range(_PARTS):
        rowv = hist_v[pl.ds(8192 + t * _HROW, _L)]
        e_parts.append(jnp.sum(jnp.where(lanes == b_c, rowv, 0)))
    o_mine = jnp.int32(0)
    for t in range(_PARTS):
        o_mine = jnp.where(part > t, o_mine + e_parts[t], o_mine)
    local_need = r_c - o_mine

    n_blk = _CHUNK // 256

    @plsc.parallel_loop(0, n_blk, unroll=2)
    def _(bi):
        acc = zero16
        for u in range(16):
            k = keys_v[pl.ds(bi * 256 + u * _L, _L)]
            acc = acc + (k == t_key).astype(jnp.int32)
        hist_v[pl.ds(bi * _L, _L)] = acc

    def d2body(bi, carry):
        cnt, blk, before = carry
        tot = jnp.sum(hist_v[pl.ds(bi * _L, _L)])
        hit = (cnt < local_need) & (cnt + tot >= local_need)
        blk = jnp.where(hit, bi, blk)
        before = jnp.where(hit, cnt, before)
        return cnt + tot, blk, before

    _, blk_star, cnt_before = lax.fori_loop(
        0, n_blk, d2body, (jnp.int32(0), jnp.int32(-1), jnp.int32(0)), unroll=4
    )
    need_blk = local_need - cnt_before
    blk_rd = jnp.maximum(blk_star, 0)

    def d3body(u, carry):
        cnt, pos = carry
        k = keys_v[pl.ds(blk_rd * 256 + u * _L, _L)]
        eq = (k == t_key).astype(jnp.int32)
        cs = plsc.cumsum(eq)
        hitl = (eq > 0) & ((cnt + cs) == need_blk)
        lpos = jnp.max(jnp.where(hitl, lanes, -1))
        pos = jnp.where(lpos >= 0, blk_rd * 256 + u * _L + lpos, pos)
        return cnt + jnp.sum(eq), pos

    _, pos = lax.fori_loop(0, 16, d3body, (jnp.int32(0), jnp.int32(-1)))
    m_cand = jnp.where(
        (blk_star >= 0) & (pos >= 0), part * _CHUNK + pos + 1, -1
    )

    merged_v[pl.ds(0, _L)] = jnp.where(lanes == part, m_cand, 0)
    plsc.subcore_barrier()
    pltpu.sync_copy(merged_v.at[pl.ds(0, _L)], shared.at[s, pl.ds(0, _L)])
    plsc.subcore_barrier()
    macc = zero16
    for t in range(_PARTS):
        pltpu.sync_copy(
            shared.at[g0 + t, pl.ds(0, _L)], hist_v.at[pl.ds(t * _HROW, _L)]
        )
    for t in range(_PARTS):
        macc = macc + hist_v[pl.ds(t * _HROW, _L)]
    m_star = jnp.max(macc)

    valid_k = k_train >= 1
    t_out = jnp.where(valid_k, t_key, jnp.int32(_SENTINEL_T))
    m_out = jnp.where(valid_k, m_star, 0)

    @pl.when(part == 0)
    def _():
        merged_v[pl.ds(0, _L)] = jnp.where(
            lanes == 0, t_out, jnp.where(lanes == 1, m_out, 0)
        )
        pltpu.sync_copy(merged_v.at[pl.ds(0, _L)], sel_hbm.at[pl.ds(img * _L, _L)])


@functools.partial(
    pl.kernel,
    mesh=plsc.VectorSubcoreMesh(core_axis_name="c", subcore_axis_name="s"),
    out_type=jax.ShapeDtypeStruct((_NIMG * _L,), jnp.int32),
    compiler_params=pltpu.CompilerParams(needs_layout_passes=False),
    scratch_types=[
        pltpu.VMEM((_CHUNK,), jnp.int32),
        pltpu.VMEM((_L * _HROW,), jnp.int32),
        pltpu.VMEM((_HROW,), jnp.int32),
        pltpu.VMEM((_WIN,), jnp.float32),
        pltpu.VMEM((_WIN,), jnp.int32),
        pltpu.VMEM_SHARED((_L, _HROW), jnp.int32),
    ],
)
def _select(pred_hbm, mask_hbm, sel_hbm, keys_v, hist_v, merged_v, winp, winm, shared):
    _sc_select(pred_hbm, mask_hbm, sel_hbm, keys_v, hist_v, merged_v, winp, winm, shared)


@jax.jit
def kernel(pred, mask):
    if pred.ndim == 4 and pred.shape[1] == 1:
        pred = pred[:, 0]
    b = pred.shape[0]
    pred3 = pred.reshape(b, _ROWS, _COLS)
    mask3 = mask.astype(jnp.int32).reshape(b, _ROWS, _COLS)

    sel = _select(
        pred.reshape(b * _PARTS, _CHUNK), mask3.reshape(b * _PARTS, _CHUNK)
    )

    train8, hold8, parts = pl.pallas_call(
        _pass_b,
        grid=(b,),
        in_specs=[
            pl.BlockSpec((1, _ROWS, _COLS), lambda i: (i, 0, 0)),
            pl.BlockSpec((1, _ROWS, _COLS), lambda i: (i, 0, 0)),
            pl.BlockSpec((1, 1, _L), lambda i: (i, 0, 0), memory_space=pltpu.SMEM),
        ],
        out_specs=[
            pl.BlockSpec((1, _ROWS, _COLS), lambda i: (i, 0, 0)),
            pl.BlockSpec((1, _ROWS, _COLS), lambda i: (i, 0, 0)),
            pl.BlockSpec((1, 1, _COLS), lambda i: (i, 0, 0)),
        ],
        out_shape=[
            jax.ShapeDtypeStruct((b, _ROWS, _COLS), jnp.bool_),
            jax.ShapeDtypeStruct((b, _ROWS, _COLS), jnp.bool_),
            jax.ShapeDtypeStruct((b, 1, _COLS), jnp.float32),
        ],
    )(pred3, mask3, sel.reshape(b, 1, _L))

    train = train8.reshape(b, 512, 512)
    hold = hold8.reshape(b, 512, 512)
    parts = parts[:, 0, :]
    num = parts[:, 0].sum()
    den = parts[:, 1].sum()
    loss = num / (den + _EPS)
    stats10 = parts[:, 2:12].sum(axis=0)
    n_holdout = stats10[9]
    n_h_correct = stats10[5] + stats10[7]
    acc = jnp.where(
        n_holdout > 0, n_h_correct / jnp.maximum(n_holdout, 1.0), 0.0
    ).astype(jnp.float32)
    stats = jnp.concatenate([stats10, acc[None]])
    return loss, train, hold, stats


# speedup vs baseline: 1.0871x; 1.0871x over previous
import functools

import jax
import jax.numpy as jnp
from jax import lax
from jax.experimental import pallas as pl
from jax.experimental.pallas import tpu as pltpu
from jax.experimental.pallas import tpu_sc as plsc

_CONF_TH = 0.85
_IGNORE = 2
_EPS = 1e-07
_DROP = 0.5

_ROWS = 2048
_COLS = 128
_N = _ROWS * _COLS

_LO1 = 0x3F400000
_SENTINEL_T = 0x41000000
_RANGE = _SENTINEL_T - _LO1

_NIMG = 8
_PARTS = 4
_CHUNK = _N // _PARTS

_SH_A = 14
_NB_A = _RANGE >> _SH_A
_SH_B = 3
_NB_B = 1 << (_SH_A - _SH_B)
_NB_C = 16

_L = 16
_HROW = 2048
_WIN = 4096


def _keys_from(p, mf):
    ann = mf != float(_IGNORE)
    conf = jnp.maximum(p, 1.0 - p)
    corr = (p > 0.5) == (mf == 1.0)
    isconf = conf > _CONF_TH
    score = jnp.ones_like(p)
    score = jnp.where(isconf & corr, 1.0, score)
    score = jnp.where((~isconf) & corr, 2.0, score)
    score = jnp.where((~isconf) & (~corr), 3.0, score)
    score = jnp.where(isconf & (~corr), 4.0, score)
    bonus = (conf - 0.5) * 0.5
    s = jnp.where(corr, score - bonus, score + bonus)
    key = jnp.where(ann, lax.bitcast_convert_type(s, jnp.int32), 0)
    return key, ann, conf, corr, isconf


def _pass_b(pred_ref, mask_ref, sel_ref, train_ref, hold_ref, part_ref):
    p = pred_ref[0]
    m = mask_ref[0]
    mf = m.astype(jnp.float32)
    key, ann, conf, corr, isconf = _keys_from(p, mf)

    t_key = sel_ref[0, 0, 0]
    m_star = sel_ref[0, 0, 1]

    rows = lax.broadcasted_iota(jnp.int32, (_ROWS, _COLS), 0)
    cols = lax.broadcasted_iota(jnp.int32, (_ROWS, _COLS), 1)
    fi = rows * _COLS + cols
    train = (key > t_key) | ((key == t_key) & (fi < m_star))
    hold = ann & (~train)

    pcl = jnp.clip(p, _EPS, 1.0 - _EPS)
    bce = -(mf * jnp.log(pcl) + (1.0 - mf) * jnp.log(1.0 - pcl))
    tf32 = train.astype(jnp.float32)
    hf32 = hold.astype(jnp.float32)

    cc = (isconf & corr).astype(jnp.float32)
    ci = (isconf & (~corr)).astype(jnp.float32)
    uc = ((~isconf) & corr).astype(jnp.float32)
    ui = ((~isconf) & (~corr)).astype(jnp.float32)

    den = jnp.sum(tf32)
    vals = [
        jnp.sum(bce * tf32),
        den,
        jnp.sum(cc * tf32),
        jnp.sum(ci * tf32),
        jnp.sum(uc * tf32),
        jnp.sum(ui * tf32),
        den,
        jnp.sum(cc * hf32),
        jnp.sum(ci * hf32),
        jnp.sum(uc * hf32),
        jnp.sum(ui * hf32),
        jnp.sum(hf32),
    ]
    col = lax.broadcasted_iota(jnp.int32, (1, _COLS), 1)
    out = jnp.zeros((1, _COLS), jnp.float32)
    for j, v in enumerate(vals):
        out = jnp.where(col == j, v, out)
    part_ref[0] = out

    train_ref[0] = train
    hold_ref[0] = hold


def _sc_select(
    pred_hbm, mask_hbm, sel_hbm, keys_v, hist_v, merged_v,
    winp0, winm0, winp1, winm1, sem0, sem1, shared,
):
    c = lax.axis_index("c")
    s = lax.axis_index("s")
    img = c * _PARTS + s // _PARTS
    part = s % _PARTS
    row = c * 16 + s
    g0 = (s // _PARTS) * _PARTS

    lanes = lax.iota(jnp.int32, _L)
    zero16 = jnp.zeros((_L,), jnp.int32)
    ones16 = jnp.ones((_L,), jnp.int32)

    def zero_hist(nb):
        @plsc.parallel_loop(0, nb, unroll=8)
        def _(i):
            hist_v[pl.ds(i * _L, _L)] = zero16

    def hist_pass(base, shift, nb, limit, low_only):
        off = lanes * nb

        @plsc.parallel_loop(0, _CHUNK // _L, unroll=8)
        def _(i):
            k = keys_v[pl.ds(i * _L, _L)]
            b = (k - base) >> shift
            if low_only:
                valid = b >= 0
            else:
                valid = (b >= 0) & (b < limit)
            plsc.addupdate_scatter(hist_v, [b + off], ones16, mask=valid)

    def collapse(nb):
        @plsc.parallel_loop(0, nb // _L, unroll=1)
        def _(j):
            acc = zero16
            for l in range(_L):
                acc = acc + hist_v[pl.ds(l * nb + j * _L, _L)]
            merged_v[pl.ds(j * _L, _L)] = acc

    def exchange_and_sum(nb, stage_off):
        plsc.subcore_barrier()
        pltpu.sync_copy(merged_v.at[pl.ds(0, nb)], shared.at[s, pl.ds(0, nb)])
        plsc.subcore_barrier()
        for t in range(_PARTS):
            pltpu.sync_copy(
                shared.at[g0 + t, pl.ds(0, nb)],
                hist_v.at[pl.ds(stage_off + t * _HROW, nb)],
            )

        @plsc.parallel_loop(0, nb // _L, unroll=2)
        def _(j):
            acc = zero16
            for t in range(_PARTS):
                acc = acc + hist_v[pl.ds(stage_off + t * _HROW + j * _L, _L)]
            merged_v[pl.ds(j * _L, _L)] = acc

    def suffix_scan(nb, need):
        def sbody(jj, carry):
            cry, bstar, rnext = carry
            j = nb // _L - 1 - jj
            v = merged_v[pl.ds(j * _L, _L)]
            q = lax.rev(plsc.cumsum(lax.rev(v, (0,))), (0,))
            suf = cry + q - v
            hit = (suf < need) & (suf + v >= need)
            bsel = jnp.max(jnp.where(hit, j * _L + lanes, -1))
            ssel = jnp.max(jnp.where(hit, suf, -1))
            bstar = jnp.where(bsel >= 0, bsel, bstar)
            rnext = jnp.where(bsel >= 0, need - ssel, rnext)
            return cry + jnp.sum(v), bstar, rnext

        _, bstar, rnext = lax.fori_loop(
            0, nb // _L, sbody, (jnp.int32(0), jnp.int32(-1), jnp.int32(-1))
        )
        return bstar, rnext

    zero_hist(_NB_A)
    off_a = lanes * _NB_A
    n_win = _CHUNK // _WIN
    bufs = ((winp0, winm0, sem0), (winp1, winm1, sem1))
    handles = [None, None]

    def start_win(w):
        bp, bm, sm = bufs[w % 2]
        hp = pltpu.async_copy(pred_hbm.at[row, pl.ds(w * _WIN, _WIN)], bp, sm)
        hm = pltpu.async_copy(mask_hbm.at[row, pl.ds(w * _WIN, _WIN)], bm, sm)
        handles[w % 2] = (hp, hm)

    start_win(0)
    for w in range(n_win):
        if w + 1 < n_win:
            start_win(w + 1)
        hp, hm = handles[w % 2]
        hp.wait()
        hm.wait()
        bp, bm, _ = bufs[w % 2]

        @plsc.parallel_loop(0, _WIN // _L, unroll=8)
        def _(i):
            p = bp[pl.ds(i * _L, _L)]
            m = bm[pl.ds(i * _L, _L)]
            conf = jnp.maximum(p, 1.0 - p)
            corr = (p > 0.5) == (m == 1)
            isconf = conf > _CONF_TH
            score = jnp.where(
                corr,
                jnp.where(isconf, 1.0, 2.0),
                jnp.where(isconf, 4.0, 3.0),
            )
            bonus = (conf - 0.5) * 0.5
            sv = jnp.where(corr, score - bonus, score + bonus)
            key = jnp.where(
                m != _IGNORE, lax.bitcast_convert_type(sv, jnp.int32), 0
            )
            keys_v[pl.ds(w * _WIN + i * _L, _L)] = key
            b = (key - _LO1) >> _SH_A
            plsc.addupdate_scatter(hist_v, [b + off_a], ones16, mask=b >= 0)

    collapse(_NB_A)
    exchange_and_sum(_NB_A, 0)

    def abody(j, acc):
        return acc + merged_v[pl.ds(j * _L, _L)]

    a_tot = jnp.sum(lax.fori_loop(0, _NB_A // _L, abody, zero16))
    k_train = a_tot >> 1

    b_a, r_a = suffix_scan(_NB_A, k_train)
    base_b = _LO1 + b_a * (1 << _SH_A)

    zero_hist(_NB_B)
    hist_pass(base_b, _SH_B, _NB_B, _NB_B, False)
    collapse(_NB_B)
    exchange_and_sum(_NB_B, 0)
    b_b, r_b = suffix_scan(_NB_B, r_a)
    base_c = base_b + b_b * (1 << _SH_B)

    zero_hist(_NB_C)
    hist_pass(base_c, 0, _NB_C, 1 << _SH_B, False)
    collapse(_NB_C)
    exchange_and_sum(_NB_C, 8192)
    b_c, r_c = suffix_scan(_NB_C, r_b)
    t_key = base_c + b_c

    e_parts = []
    for t in range(_PARTS):
        rowv = hist_v[pl.ds(8192 + t * _HROW, _L)]
        e_parts.append(jnp.sum(jnp.where(lanes == b_c, rowv, 0)))
    o_mine = jnp.int32(0)
    for t in range(_PARTS):
        o_mine = jnp.where(part > t, o_mine + e_parts[t], o_mine)
    local_need = r_c - o_mine

    n_blk = _CHUNK // 256

    @plsc.parallel_loop(0, n_blk, unroll=2)
    def _(bi):
        acc = zero16
        for u in range(16):
            k = keys_v[pl.ds(bi * 256 + u * _L, _L)]
            acc = acc + (k == t_key).astype(jnp.int32)
        hist_v[pl.ds(bi * _L, _L)] = acc

    def d2body(bi, carry):
        cnt, blk, before = carry
        tot = jnp.sum(hist_v[pl.ds(bi * _L, _L)])
        hit = (cnt < local_need) & (cnt + tot >= local_need)
        blk = jnp.where(hit, bi, blk)
        before = jnp.where(hit, cnt, before)
        return cnt + tot, blk, before

    _, blk_star, cnt_before = lax.fori_loop(
        0, n_blk, d2body, (jnp.int32(0), jnp.int32(-1), jnp.int32(0)), unroll=4
    )
    need_blk = local_need - cnt_before
    blk_rd = jnp.maximum(blk_star, 0)

    def d3body(u, carry):
        cnt, pos = carry
        k = keys_v[pl.ds(blk_rd * 256 + u * _L, _L)]
        eq = (k == t_key).astype(jnp.int32)
        cs = plsc.cumsum(eq)
        hitl = (eq > 0) & ((cnt + cs) == need_blk)
        lpos = jnp.max(jnp.where(hitl, lanes, -1))
        pos = jnp.where(lpos >= 0, blk_rd * 256 + u * _L + lpos, pos)
        return cnt + jnp.sum(eq), pos

    _, pos = lax.fori_loop(0, 16, d3body, (jnp.int32(0), jnp.int32(-1)))
    m_cand = jnp.where(
        (blk_star >= 0) & (pos >= 0), part * _CHUNK + pos + 1, -1
    )

    merged_v[pl.ds(0, _L)] = jnp.where(lanes == part, m_cand, 0)
    plsc.subcore_barrier()
    pltpu.sync_copy(merged_v.at[pl.ds(0, _L)], shared.at[s, pl.ds(0, _L)])
    plsc.subcore_barrier()
    macc = zero16
    for t in range(_PARTS):
        pltpu.sync_copy(
            shared.at[g0 + t, pl.ds(0, _L)], hist_v.at[pl.ds(t * _HROW, _L)]
        )
    for t in range(_PARTS):
        macc = macc + hist_v[pl.ds(t * _HROW, _L)]
    m_star = jnp.max(macc)

    valid_k = k_train >= 1
    t_out = jnp.where(valid_k, t_key, jnp.int32(_SENTINEL_T))
    m_out = jnp.where(valid_k, m_star, 0)

    @pl.when(part == 0)
    def _():
        merged_v[pl.ds(0, _L)] = jnp.where(
            lanes == 0, t_out, jnp.where(lanes == 1, m_out, 0)
        )
        pltpu.sync_copy(merged_v.at[pl.ds(0, _L)], sel_hbm.at[pl.ds(img * _L, _L)])


@functools.partial(
    pl.kernel,
    mesh=plsc.VectorSubcoreMesh(core_axis_name="c", subcore_axis_name="s"),
    out_type=jax.ShapeDtypeStruct((_NIMG * _L,), jnp.int32),
    compiler_params=pltpu.CompilerParams(needs_layout_passes=False),
    scratch_types=[
        pltpu.VMEM((_CHUNK,), jnp.int32),
        pltpu.VMEM((_L * _HROW,), jnp.int32),
        pltpu.VMEM((_HROW,), jnp.int32),
        pltpu.VMEM((_WIN,), jnp.float32),
        pltpu.VMEM((_WIN,), jnp.int32),
        pltpu.VMEM((_WIN,), jnp.float32),
        pltpu.VMEM((_WIN,), jnp.int32),
        pltpu.SemaphoreType.DMA,
        pltpu.SemaphoreType.DMA,
        pltpu.VMEM_SHARED((_L, _HROW), jnp.int32),
    ],
)
def _select(
    pred_hbm, mask_hbm, sel_hbm, keys_v, hist_v, merged_v,
    winp0, winm0, winp1, winm1, sem0, sem1, shared,
):
    _sc_select(
        pred_hbm, mask_hbm, sel_hbm, keys_v, hist_v, merged_v,
        winp0, winm0, winp1, winm1, sem0, sem1, shared,
    )


@jax.jit
def kernel(pred, mask):
    if pred.ndim == 4 and pred.shape[1] == 1:
        pred = pred[:, 0]
    b = pred.shape[0]
    pred3 = pred.reshape(b, _ROWS, _COLS)
    mask3 = mask.astype(jnp.int32).reshape(b, _ROWS, _COLS)

    sel = _select(
        pred.reshape(b * _PARTS, _CHUNK), mask3.reshape(b * _PARTS, _CHUNK)
    )

    train8, hold8, parts = pl.pallas_call(
        _pass_b,
        grid=(b,),
        in_specs=[
            pl.BlockSpec((1, _ROWS, _COLS), lambda i: (i, 0, 0)),
            pl.BlockSpec((1, _ROWS, _COLS), lambda i: (i, 0, 0)),
            pl.BlockSpec((1, 1, _L), lambda i: (i, 0, 0), memory_space=pltpu.SMEM),
        ],
        out_specs=[
            pl.BlockSpec((1, _ROWS, _COLS), lambda i: (i, 0, 0)),
            pl.BlockSpec((1, _ROWS, _COLS), lambda i: (i, 0, 0)),
            pl.BlockSpec((1, 1, _COLS), lambda i: (i, 0, 0)),
        ],
        out_shape=[
            jax.ShapeDtypeStruct((b, _ROWS, _COLS), jnp.bool_),
            jax.ShapeDtypeStruct((b, _ROWS, _COLS), jnp.bool_),
            jax.ShapeDtypeStruct((b, 1, _COLS), jnp.float32),
        ],
    )(pred3, mask3, sel.reshape(b, 1, _L))

    train = train8.reshape(b, 512, 512)
    hold = hold8.reshape(b, 512, 512)
    parts = parts[:, 0, :]
    num = parts[:, 0].sum()
    den = parts[:, 1].sum()
    loss = num / (den + _EPS)
    stats10 = parts[:, 2:12].sum(axis=0)
    n_holdout = stats10[9]
    n_h_correct = stats10[5] + stats10[7]
    acc = jnp.where(
        n_holdout > 0, n_h_correct / jnp.maximum(n_holdout, 1.0), 0.0
    ).astype(jnp.float32)
    stats = jnp.concatenate([stats10, acc[None]])
    return loss, train, hold, stats


# speedup vs baseline: 1.1377x; 1.0466x over previous
import functools

import jax
import jax.numpy as jnp
from jax import lax
from jax.experimental import pallas as pl
from jax.experimental.pallas import tpu as pltpu
from jax.experimental.pallas import tpu_sc as plsc

_CONF_TH = 0.85
_IGNORE = 2
_EPS = 1e-07
_DROP = 0.5

_ROWS = 2048
_COLS = 128
_N = _ROWS * _COLS

_LO1 = 0x3F400000
_SENTINEL_T = 0x41000000
_RANGE = _SENTINEL_T - _LO1

_NIMG = 8
_PARTS = 4
_CHUNK = _N // _PARTS

_SH_A = 14
_NB_A = _RANGE >> _SH_A
_SH_B = 3
_NB_B = 1 << (_SH_A - _SH_B)
_NB_C = 16

_L = 16
_HROW = 2048
_WIN = 4096


def _keys_from(p, mf):
    ann = mf != float(_IGNORE)
    conf = jnp.maximum(p, 1.0 - p)
    corr = (p > 0.5) == (mf == 1.0)
    isconf = conf > _CONF_TH
    score = jnp.ones_like(p)
    score = jnp.where(isconf & corr, 1.0, score)
    score = jnp.where((~isconf) & corr, 2.0, score)
    score = jnp.where((~isconf) & (~corr), 3.0, score)
    score = jnp.where(isconf & (~corr), 4.0, score)
    bonus = (conf - 0.5) * 0.5
    s = jnp.where(corr, score - bonus, score + bonus)
    key = jnp.where(ann, lax.bitcast_convert_type(s, jnp.int32), 0)
    return key, ann, conf, corr, isconf


def _pass_b(pred_ref, mask_ref, sel_ref, train_ref, hold_ref, part_ref):
    p = pred_ref[0]
    m = mask_ref[0]
    mf = m.astype(jnp.float32)
    key, ann, conf, corr, isconf = _keys_from(p, mf)

    t_key = sel_ref[0, 0, 0]
    m_star = sel_ref[0, 0, 1]

    rows = lax.broadcasted_iota(jnp.int32, (_ROWS, _COLS), 0)
    cols = lax.broadcasted_iota(jnp.int32, (_ROWS, _COLS), 1)
    fi = rows * _COLS + cols
    train = (key > t_key) | ((key == t_key) & (fi < m_star))
    hold = ann & (~train)

    pcl = jnp.clip(p, _EPS, 1.0 - _EPS)
    bce = -(mf * jnp.log(pcl) + (1.0 - mf) * jnp.log(1.0 - pcl))
    tf32 = train.astype(jnp.float32)
    hf32 = hold.astype(jnp.float32)

    cc = (isconf & corr).astype(jnp.float32)
    ci = (isconf & (~corr)).astype(jnp.float32)
    uc = ((~isconf) & corr).astype(jnp.float32)
    ui = ((~isconf) & (~corr)).astype(jnp.float32)

    den = jnp.sum(tf32)
    vals = [
        jnp.sum(bce * tf32),
        den,
        jnp.sum(cc * tf32),
        jnp.sum(ci * tf32),
        jnp.sum(uc * tf32),
        jnp.sum(ui * tf32),
        den,
        jnp.sum(cc * hf32),
        jnp.sum(ci * hf32),
        jnp.sum(uc * hf32),
        jnp.sum(ui * hf32),
        jnp.sum(hf32),
    ]
    col = lax.broadcasted_iota(jnp.int32, (1, _COLS), 1)
    out = jnp.zeros((1, _COLS), jnp.float32)
    for j, v in enumerate(vals):
        out = jnp.where(col == j, v, out)

    step = pl.program_id(0)
    prev = jnp.where(step == 0, jnp.zeros((1, _COLS), jnp.float32), part_ref[0])
    total = prev + out
    part_ref[0] = total

    @pl.when(step == pl.num_programs(0) - 1)
    def _():
        def pick(j):
            return jnp.sum(jnp.where(col == j, total, 0.0))

        num = pick(0)
        den_t = pick(1)
        loss = num / (den_t + _EPS)
        n_holdout = pick(11)
        n_h_correct = pick(7) + pick(9)
        acc = jnp.where(
            n_holdout > 0, n_h_correct / jnp.maximum(n_holdout, 1.0), 0.0
        )
        fin = jnp.where(col == 0, loss, 0.0)
        for j in range(10):
            fin = jnp.where(col == j + 1, pick(j + 2), fin)
        fin = jnp.where(col == 11, acc, fin)
        part_ref[0] = fin

    train_ref[0] = train
    hold_ref[0] = hold


def _sc_select(
    pred_hbm, mask_hbm, sel_hbm, keys_v, hist_v, merged_v,
    winp0, winm0, winp1, winm1, sem0, sem1, shared,
):
    c = lax.axis_index("c")
    s = lax.axis_index("s")
    img = c * _PARTS + s // _PARTS
    part = s % _PARTS
    row = c * 16 + s
    g0 = (s // _PARTS) * _PARTS

    lanes = lax.iota(jnp.int32, _L)
    zero16 = jnp.zeros((_L,), jnp.int32)
    ones16 = jnp.ones((_L,), jnp.int32)

    def zero_hist(nb):
        @plsc.parallel_loop(0, nb, unroll=8)
        def _(i):
            hist_v[pl.ds(i * _L, _L)] = zero16

    def hist_pass(base, shift, nb, limit, low_only):
        off = lanes * nb

        @plsc.parallel_loop(0, _CHUNK // _L, unroll=8)
        def _(i):
            k = keys_v[pl.ds(i * _L, _L)]
            b = (k - base) >> shift
            if low_only:
                valid = b >= 0
            else:
                valid = (b >= 0) & (b < limit)
            plsc.addupdate_scatter(hist_v, [b + off], ones16, mask=valid)

    def collapse(nb):
        @plsc.parallel_loop(0, nb // _L, unroll=1)
        def _(j):
            acc = zero16
            for l in range(_L):
                acc = acc + hist_v[pl.ds(l * nb + j * _L, _L)]
            merged_v[pl.ds(j * _L, _L)] = acc

    def exchange_and_sum(nb, stage_off):
        plsc.subcore_barrier()
        pltpu.sync_copy(merged_v.at[pl.ds(0, nb)], shared.at[s, pl.ds(0, nb)])
        plsc.subcore_barrier()
        for t in range(_PARTS):
            pltpu.sync_copy(
                shared.at[g0 + t, pl.ds(0, nb)],
                hist_v.at[pl.ds(stage_off + t * _HROW, nb)],
            )

        @plsc.parallel_loop(0, nb // _L, unroll=2)
        def _(j):
            acc = zero16
            for t in range(_PARTS):
                acc = acc + hist_v[pl.ds(stage_off + t * _HROW + j * _L, _L)]
            merged_v[pl.ds(j * _L, _L)] = acc

    def suffix_scan(nb, need):
        def sbody(jj, carry):
            cry, bstar, rnext = carry
            j = nb // _L - 1 - jj
            v = merged_v[pl.ds(j * _L, _L)]
            q = lax.rev(plsc.cumsum(lax.rev(v, (0,))), (0,))
            suf = cry + q - v
            hit = (suf < need) & (suf + v >= need)
            bsel = jnp.max(jnp.where(hit, j * _L + lanes, -1))
            ssel = jnp.max(jnp.where(hit, suf, -1))
            bstar = jnp.where(bsel >= 0, bsel, bstar)
            rnext = jnp.where(bsel >= 0, need - ssel, rnext)
            return cry + jnp.sum(v), bstar, rnext

        _, bstar, rnext = lax.fori_loop(
            0, nb // _L, sbody, (jnp.int32(0), jnp.int32(-1), jnp.int32(-1))
        )
        return bstar, rnext

    zero_hist(_NB_A)
    off_a = lanes * _NB_A
    n_win = _CHUNK // _WIN
    bufs = ((winp0, winm0, sem0), (winp1, winm1, sem1))
    handles = [None, None]

    def start_win(w):
        bp, bm, sm = bufs[w % 2]
        hp = pltpu.async_copy(pred_hbm.at[row, pl.ds(w * _WIN, _WIN)], bp, sm)
        hm = pltpu.async_copy(mask_hbm.at[row, pl.ds(w * _WIN, _WIN)], bm, sm)
        handles[w % 2] = (hp, hm)

    start_win(0)
    for w in range(n_win):
        if w + 1 < n_win:
            start_win(w + 1)
        hp, hm = handles[w % 2]
        hp.wait()
        hm.wait()
        bp, bm, _ = bufs[w % 2]

        @plsc.parallel_loop(0, _WIN // _L, unroll=8)
        def _(i):
            p = bp[pl.ds(i * _L, _L)]
            m = bm[pl.ds(i * _L, _L)]
            conf = jnp.maximum(p, 1.0 - p)
            corr = (p > 0.5) == (m == 1)
            isconf = conf > _CONF_TH
            score = jnp.where(
                corr,
                jnp.where(isconf, 1.0, 2.0),
                jnp.where(isconf, 4.0, 3.0),
            )
            bonus = (conf - 0.5) * 0.5
            sv = jnp.where(corr, score - bonus, score + bonus)
            key = jnp.where(
                m != _IGNORE, lax.bitcast_convert_type(sv, jnp.int32), 0
            )
            keys_v[pl.ds(w * _WIN + i * _L, _L)] = key
            b = (key - _LO1) >> _SH_A
            plsc.addupdate_scatter(hist_v, [b + off_a], ones16, mask=b >= 0)

    collapse(_NB_A)
    exchange_and_sum(_NB_A, 0)

    def abody(j, acc):
        return acc + merged_v[pl.ds(j * _L, _L)]

    a_tot = jnp.sum(lax.fori_loop(0, _NB_A // _L, abody, zero16))
    k_train = a_tot >> 1

    b_a, r_a = suffix_scan(_NB_A, k_train)
    base_b = _LO1 + b_a * (1 << _SH_A)

    zero_hist(_NB_B)
    hist_pass(base_b, _SH_B, _NB_B, _NB_B, False)
    collapse(_NB_B)
    exchange_and_sum(_NB_B, 0)
    b_b, r_b = suffix_scan(_NB_B, r_a)
    base_c = base_b + b_b * (1 << _SH_B)

    zero_hist(_NB_C)
    hist_pass(base_c, 0, _NB_C, 1 << _SH_B, False)
    collapse(_NB_C)
    exchange_and_sum(_NB_C, 8192)
    b_c, r_c = suffix_scan(_NB_C, r_b)
    t_key = base_c + b_c

    e_parts = []
    for t in range(_PARTS):
        rowv = hist_v[pl.ds(8192 + t * _HROW, _L)]
        e_parts.append(jnp.sum(jnp.where(lanes == b_c, rowv, 0)))
    o_mine = jnp.int32(0)
    for t in range(_PARTS):
        o_mine = jnp.where(part > t, o_mine + e_parts[t], o_mine)
    local_need = r_c - o_mine

    n_blk = _CHUNK // 256

    @plsc.parallel_loop(0, n_blk, unroll=2)
    def _(bi):
        acc = zero16
        for u in range(16):
            k = keys_v[pl.ds(bi * 256 + u * _L, _L)]
            acc = acc + (k == t_key).astype(jnp.int32)
        hist_v[pl.ds(bi * _L, _L)] = acc

    def d2body(bi, carry):
        cnt, blk, before = carry
        tot = jnp.sum(hist_v[pl.ds(bi * _L, _L)])
        hit = (cnt < local_need) & (cnt + tot >= local_need)
        blk = jnp.where(hit, bi, blk)
        before = jnp.where(hit, cnt, before)
        return cnt + tot, blk, before

    _, blk_star, cnt_before = lax.fori_loop(
        0, n_blk, d2body, (jnp.int32(0), jnp.int32(-1), jnp.int32(0)), unroll=4
    )
    need_blk = local_need - cnt_before
    blk_rd = jnp.maximum(blk_star, 0)

    def d3body(u, carry):
        cnt, pos = carry
        k = keys_v[pl.ds(blk_rd * 256 + u * _L, _L)]
        eq = (k == t_key).astype(jnp.int32)
        cs = plsc.cumsum(eq)
        hitl = (eq > 0) & ((cnt + cs) == need_blk)
        lpos = jnp.max(jnp.where(hitl, lanes, -1))
        pos = jnp.where(lpos >= 0, blk_rd * 256 + u * _L + lpos, pos)
        return cnt + jnp.sum(eq), pos

    _, pos = lax.fori_loop(0, 16, d3body, (jnp.int32(0), jnp.int32(-1)))
    m_cand = jnp.where(
        (blk_star >= 0) & (pos >= 0), part * _CHUNK + pos + 1, -1
    )

    merged_v[pl.ds(0, _L)] = jnp.where(lanes == part, m_cand, 0)
    plsc.subcore_barrier()
    pltpu.sync_copy(merged_v.at[pl.ds(0, _L)], shared.at[s, pl.ds(0, _L)])
    plsc.subcore_barrier()
    macc = zero16
    for t in range(_PARTS):
        pltpu.sync_copy(
            shared.at[g0 + t, pl.ds(0, _L)], hist_v.at[pl.ds(t * _HROW, _L)]
        )
    for t in range(_PARTS):
        macc = macc + hist_v[pl.ds(t * _HROW, _L)]
    m_star = jnp.max(macc)

    valid_k = k_train >= 1
    t_out = jnp.where(valid_k, t_key, jnp.int32(_SENTINEL_T))
    m_out = jnp.where(valid_k, m_star, 0)

    @pl.when(part == 0)
    def _():
        merged_v[pl.ds(0, _L)] = jnp.where(
            lanes == 0, t_out, jnp.where(lanes == 1, m_out, 0)
        )
        pltpu.sync_copy(merged_v.at[pl.ds(0, _L)], sel_hbm.at[pl.ds(img * _L, _L)])


@functools.partial(
    pl.kernel,
    mesh=plsc.VectorSubcoreMesh(core_axis_name="c", subcore_axis_name="s"),
    out_type=jax.ShapeDtypeStruct((_NIMG * _L,), jnp.int32),
    compiler_params=pltpu.CompilerParams(needs_layout_passes=False),
    scratch_types=[
        pltpu.VMEM((_CHUNK,), jnp.int32),
        pltpu.VMEM((_L * _HROW,), jnp.int32),
        pltpu.VMEM((_HROW,), jnp.int32),
        pltpu.VMEM((_WIN,), jnp.float32),
        pltpu.VMEM((_WIN,), jnp.int32),
        pltpu.VMEM((_WIN,), jnp.float32),
        pltpu.VMEM((_WIN,), jnp.int32),
        pltpu.SemaphoreType.DMA,
        pltpu.SemaphoreType.DMA,
        pltpu.VMEM_SHARED((_L, _HROW), jnp.int32),
    ],
)
def _select(
    pred_hbm, mask_hbm, sel_hbm, keys_v, hist_v, merged_v,
    winp0, winm0, winp1, winm1, sem0, sem1, shared,
):
    _sc_select(
        pred_hbm, mask_hbm, sel_hbm, keys_v, hist_v, merged_v,
        winp0, winm0, winp1, winm1, sem0, sem1, shared,
    )


@jax.jit
def kernel(pred, mask):
    if pred.ndim == 4 and pred.shape[1] == 1:
        pred = pred[:, 0]
    b = pred.shape[0]
    pred3 = pred.reshape(b, _ROWS, _COLS)
    mask3 = mask.astype(jnp.int32).reshape(b, _ROWS, _COLS)

    sel = _select(
        pred.reshape(b * _PARTS, _CHUNK), mask3.reshape(b * _PARTS, _CHUNK)
    )

    train8, hold8, parts = pl.pallas_call(
        _pass_b,
        grid=(b,),
        in_specs=[
            pl.BlockSpec((1, _ROWS, _COLS), lambda i: (i, 0, 0)),
            pl.BlockSpec((1, _ROWS, _COLS), lambda i: (i, 0, 0)),
            pl.BlockSpec((1, 1, _L), lambda i: (i, 0, 0), memory_space=pltpu.SMEM),
        ],
        out_specs=[
            pl.BlockSpec((1, _ROWS, _COLS), lambda i: (i, 0, 0)),
            pl.BlockSpec((1, _ROWS, _COLS), lambda i: (i, 0, 0)),
            pl.BlockSpec((1, 1, _COLS), lambda i: (0, 0, 0)),
        ],
        out_shape=[
            jax.ShapeDtypeStruct((b, _ROWS, _COLS), jnp.bool_),
            jax.ShapeDtypeStruct((b, _ROWS, _COLS), jnp.bool_),
            jax.ShapeDtypeStruct((1, 1, _COLS), jnp.float32),
        ],
    )(pred3, mask3, sel.reshape(b, 1, _L))

    train = train8.reshape(b, 512, 512)
    hold = hold8.reshape(b, 512, 512)
    fin = parts[0, 0]
    loss = fin[0]
    stats = fin[1:12]
    return loss, train, hold, stats


# speedup vs baseline: 1.2636x; 1.1107x over previous
import functools

import jax
import jax.numpy as jnp
from jax import lax
from jax.experimental import pallas as pl
from jax.experimental.pallas import tpu as pltpu
from jax.experimental.pallas import tpu_sc as plsc

_CONF_TH = 0.85
_IGNORE = 2
_EPS = 1e-07
_DROP = 0.5

_ROWS = 2048
_COLS = 128
_N = _ROWS * _COLS

_LO1 = 0x3F400000
_SENTINEL_T = 0x41000000
_RANGE = _SENTINEL_T - _LO1

_NIMG = 8
_PARTS = 4
_CHUNK = _N // _PARTS

_SH_A = 14
_NB_A = _RANGE >> _SH_A
_SH_B = 3
_NB_B = 1 << (_SH_A - _SH_B)
_NB_C = 16

_L = 16
_HROW = 2048
_WIN = 4096


def _keys_from(p, mf):
    ann = mf != float(_IGNORE)
    conf = jnp.maximum(p, 1.0 - p)
    corr = (p > 0.5) == (mf == 1.0)
    isconf = conf > _CONF_TH
    score = jnp.ones_like(p)
    score = jnp.where(isconf & corr, 1.0, score)
    score = jnp.where((~isconf) & corr, 2.0, score)
    score = jnp.where((~isconf) & (~corr), 3.0, score)
    score = jnp.where(isconf & (~corr), 4.0, score)
    bonus = (conf - 0.5) * 0.5
    s = jnp.where(corr, score - bonus, score + bonus)
    key = jnp.where(ann, lax.bitcast_convert_type(s, jnp.int32), 0)
    return key, ann, conf, corr, isconf


def _pass_p(pred_ref, mask_ref, key_ref, bce_ref, cls_ref):
    p = pred_ref[0]
    m = mask_ref[0]
    mf = m.astype(jnp.float32)
    key, ann, conf, corr, isconf = _keys_from(p, mf)
    key_ref[0] = key
    pcl = jnp.clip(p, _EPS, 1.0 - _EPS)
    bce_ref[0] = -(mf * jnp.log(pcl) + (1.0 - mf) * jnp.log(1.0 - pcl))
    cls_ref[0] = (
        isconf.astype(jnp.int32) * 2 + corr.astype(jnp.int32)
    ).astype(jnp.int8)


def _pass_b(key_ref, bce_ref, cls_ref, sel_ref, train_ref, hold_ref, part_ref):
    key = key_ref[0]
    bce = bce_ref[0]
    cls = cls_ref[0].astype(jnp.int32)
    ann = key != 0
    corr = (cls == 1) | (cls == 3)
    isconf = cls >= 2

    t_key = sel_ref[0, 0, 0]
    m_star = sel_ref[0, 0, 1]

    rows = lax.broadcasted_iota(jnp.int32, (_ROWS, _COLS), 0)
    cols = lax.broadcasted_iota(jnp.int32, (_ROWS, _COLS), 1)
    fi = rows * _COLS + cols
    train = (key > t_key) | ((key == t_key) & (fi < m_star))
    hold = ann & (~train)

    tf32 = train.astype(jnp.float32)
    hf32 = hold.astype(jnp.float32)

    cc = (isconf & corr).astype(jnp.float32)
    ci = (isconf & (~corr)).astype(jnp.float32)
    uc = ((~isconf) & corr).astype(jnp.float32)
    ui = ((~isconf) & (~corr)).astype(jnp.float32)

    den = jnp.sum(tf32)
    vals = [
        jnp.sum(bce * tf32),
        den,
        jnp.sum(cc * tf32),
        jnp.sum(ci * tf32),
        jnp.sum(uc * tf32),
        jnp.sum(ui * tf32),
        den,
        jnp.sum(cc * hf32),
        jnp.sum(ci * hf32),
        jnp.sum(uc * hf32),
        jnp.sum(ui * hf32),
        jnp.sum(hf32),
    ]
    col = lax.broadcasted_iota(jnp.int32, (1, _COLS), 1)
    out = jnp.zeros((1, _COLS), jnp.float32)
    for j, v in enumerate(vals):
        out = jnp.where(col == j, v, out)

    step = pl.program_id(0)
    prev = jnp.where(step == 0, jnp.zeros((1, _COLS), jnp.float32), part_ref[0])
    total = prev + out
    part_ref[0] = total

    @pl.when(step == pl.num_programs(0) - 1)
    def _():
        def pick(j):
            return jnp.sum(jnp.where(col == j, total, 0.0))

        num = pick(0)
        den_t = pick(1)
        loss = num / (den_t + _EPS)
        n_holdout = pick(11)
        n_h_correct = pick(7) + pick(9)
        acc = jnp.where(
            n_holdout > 0, n_h_correct / jnp.maximum(n_holdout, 1.0), 0.0
        )
        fin = jnp.where(col == 0, loss, 0.0)
        for j in range(10):
            fin = jnp.where(col == j + 1, pick(j + 2), fin)
        fin = jnp.where(col == 11, acc, fin)
        part_ref[0] = fin

    train_ref[0] = train
    hold_ref[0] = hold


def _sc_select(
    pred_hbm, mask_hbm, sel_hbm, keys_v, hist_v, merged_v,
    winp0, winm0, winp1, winm1, sem0, sem1, shared,
):
    c = lax.axis_index("c")
    s = lax.axis_index("s")
    img = c * _PARTS + s // _PARTS
    part = s % _PARTS
    row = c * 16 + s
    g0 = (s // _PARTS) * _PARTS

    lanes = lax.iota(jnp.int32, _L)
    zero16 = jnp.zeros((_L,), jnp.int32)
    ones16 = jnp.ones((_L,), jnp.int32)

    def zero_hist(nb):
        @plsc.parallel_loop(0, nb, unroll=8)
        def _(i):
            hist_v[pl.ds(i * _L, _L)] = zero16

    def hist_pass(base, shift, nb, limit, low_only):
        off = lanes * nb

        @plsc.parallel_loop(0, _CHUNK // _L, unroll=8)
        def _(i):
            k = keys_v[pl.ds(i * _L, _L)]
            b = (k - base) >> shift
            if low_only:
                valid = b >= 0
            else:
                valid = (b >= 0) & (b < limit)
            plsc.addupdate_scatter(hist_v, [b + off], ones16, mask=valid)

    def collapse(nb):
        @plsc.parallel_loop(0, nb // _L, unroll=1)
        def _(j):
            acc = zero16
            for l in range(_L):
                acc = acc + hist_v[pl.ds(l * nb + j * _L, _L)]
            merged_v[pl.ds(j * _L, _L)] = acc

    def exchange_and_sum(nb, stage_off):
        plsc.subcore_barrier()
        pltpu.sync_copy(merged_v.at[pl.ds(0, nb)], shared.at[s, pl.ds(0, nb)])
        plsc.subcore_barrier()
        for t in range(_PARTS):
            pltpu.sync_copy(
                shared.at[g0 + t, pl.ds(0, nb)],
                hist_v.at[pl.ds(stage_off + t * _HROW, nb)],
            )

        @plsc.parallel_loop(0, nb // _L, unroll=2)
        def _(j):
            acc = zero16
            for t in range(_PARTS):
                acc = acc + hist_v[pl.ds(stage_off + t * _HROW + j * _L, _L)]
            merged_v[pl.ds(j * _L, _L)] = acc

    def suffix_scan(nb, need):
        def sbody(jj, carry):
            cry, bstar, rnext = carry
            j = nb // _L - 1 - jj
            v = merged_v[pl.ds(j * _L, _L)]
            q = lax.rev(plsc.cumsum(lax.rev(v, (0,))), (0,))
            suf = cry + q - v
            hit = (suf < need) & (suf + v >= need)
            bsel = jnp.max(jnp.where(hit, j * _L + lanes, -1))
            ssel = jnp.max(jnp.where(hit, suf, -1))
            bstar = jnp.where(bsel >= 0, bsel, bstar)
            rnext = jnp.where(bsel >= 0, need - ssel, rnext)
            return cry + jnp.sum(v), bstar, rnext

        _, bstar, rnext = lax.fori_loop(
            0, nb // _L, sbody, (jnp.int32(0), jnp.int32(-1), jnp.int32(-1))
        )
        return bstar, rnext

    zero_hist(_NB_A)
    off_a = lanes * _NB_A
    n_win = _CHUNK // _WIN
    bufs = ((winp0, winm0, sem0), (winp1, winm1, sem1))
    handles = [None, None]

    def start_win(w):
        bp, bm, sm = bufs[w % 2]
        hp = pltpu.async_copy(pred_hbm.at[row, pl.ds(w * _WIN, _WIN)], bp, sm)
        hm = pltpu.async_copy(mask_hbm.at[row, pl.ds(w * _WIN, _WIN)], bm, sm)
        handles[w % 2] = (hp, hm)

    start_win(0)
    for w in range(n_win):
        if w + 1 < n_win:
            start_win(w + 1)
        hp, hm = handles[w % 2]
        hp.wait()
        hm.wait()
        bp, bm, _ = bufs[w % 2]

        @plsc.parallel_loop(0, _WIN // _L, unroll=8)
        def _(i):
            p = bp[pl.ds(i * _L, _L)]
            m = bm[pl.ds(i * _L, _L)]
            conf = jnp.maximum(p, 1.0 - p)
            corr = (p > 0.5) == (m == 1)
            isconf = conf > _CONF_TH
            score = jnp.where(
                corr,
                jnp.where(isconf, 1.0, 2.0),
                jnp.where(isconf, 4.0, 3.0),
            )
            bonus = (conf - 0.5) * 0.5
            sv = jnp.where(corr, score - bonus, score + bonus)
            key = jnp.where(
                m != _IGNORE, lax.bitcast_convert_type(sv, jnp.int32), 0
            )
            keys_v[pl.ds(w * _WIN + i * _L, _L)] = key
            b = (key - _LO1) >> _SH_A
            plsc.addupdate_scatter(hist_v, [b + off_a], ones16, mask=b >= 0)

    collapse(_NB_A)
    exchange_and_sum(_NB_A, 0)

    def abody(j, acc):
        return acc + merged_v[pl.ds(j * _L, _L)]

    a_tot = jnp.sum(lax.fori_loop(0, _NB_A // _L, abody, zero16))
    k_train = a_tot >> 1

    b_a, r_a = suffix_scan(_NB_A, k_train)
    base_b = _LO1 + b_a * (1 << _SH_A)

    zero_hist(_NB_B)
    hist_pass(base_b, _SH_B, _NB_B, _NB_B, False)
    collapse(_NB_B)
    exchange_and_sum(_NB_B, 0)
    b_b, r_b = suffix_scan(_NB_B, r_a)
    base_c = base_b + b_b * (1 << _SH_B)

    zero_hist(_NB_C)
    hist_pass(base_c, 0, _NB_C, 1 << _SH_B, False)
    collapse(_NB_C)
    exchange_and_sum(_NB_C, 8192)
    b_c, r_c = suffix_scan(_NB_C, r_b)
    t_key = base_c + b_c

    e_parts = []
    for t in range(_PARTS):
        rowv = hist_v[pl.ds(8192 + t * _HROW, _L)]
        e_parts.append(jnp.sum(jnp.where(lanes == b_c, rowv, 0)))
    o_mine = jnp.int32(0)
    for t in range(_PARTS):
        o_mine = jnp.where(part > t, o_mine + e_parts[t], o_mine)
    local_need = r_c - o_mine

    n_blk = _CHUNK // 256

    @plsc.parallel_loop(0, n_blk, unroll=2)
    def _(bi):
        acc = zero16
        for u in range(16):
            k = keys_v[pl.ds(bi * 256 + u * _L, _L)]
            acc = acc + (k == t_key).astype(jnp.int32)
        hist_v[pl.ds(bi * _L, _L)] = acc

    def d2body(bi, carry):
        cnt, blk, before = carry
        tot = jnp.sum(hist_v[pl.ds(bi * _L, _L)])
        hit = (cnt < local_need) & (cnt + tot >= local_need)
        blk = jnp.where(hit, bi, blk)
        before = jnp.where(hit, cnt, before)
        return cnt + tot, blk, before

    _, blk_star, cnt_before = lax.fori_loop(
        0, n_blk, d2body, (jnp.int32(0), jnp.int32(-1), jnp.int32(0)), unroll=4
    )
    need_blk = local_need - cnt_before
    blk_rd = jnp.maximum(blk_star, 0)

    def d3body(u, carry):
        cnt, pos = carry
        k = keys_v[pl.ds(blk_rd * 256 + u * _L, _L)]
        eq = (k == t_key).astype(jnp.int32)
        cs = plsc.cumsum(eq)
        hitl = (eq > 0) & ((cnt + cs) == need_blk)
        lpos = jnp.max(jnp.where(hitl, lanes, -1))
        pos = jnp.where(lpos >= 0, blk_rd * 256 + u * _L + lpos, pos)
        return cnt + jnp.sum(eq), pos

    _, pos = lax.fori_loop(0, 16, d3body, (jnp.int32(0), jnp.int32(-1)))
    m_cand = jnp.where(
        (blk_star >= 0) & (pos >= 0), part * _CHUNK + pos + 1, -1
    )

    merged_v[pl.ds(0, _L)] = jnp.where(lanes == part, m_cand, 0)
    plsc.subcore_barrier()
    pltpu.sync_copy(merged_v.at[pl.ds(0, _L)], shared.at[s, pl.ds(0, _L)])
    plsc.subcore_barrier()
    macc = zero16
    for t in range(_PARTS):
        pltpu.sync_copy(
            shared.at[g0 + t, pl.ds(0, _L)], hist_v.at[pl.ds(t * _HROW, _L)]
        )
    for t in range(_PARTS):
        macc = macc + hist_v[pl.ds(t * _HROW, _L)]
    m_star = jnp.max(macc)

    valid_k = k_train >= 1
    t_out = jnp.where(valid_k, t_key, jnp.int32(_SENTINEL_T))
    m_out = jnp.where(valid_k, m_star, 0)

    @pl.when(part == 0)
    def _():
        merged_v[pl.ds(0, _L)] = jnp.where(
            lanes == 0, t_out, jnp.where(lanes == 1, m_out, 0)
        )
        pltpu.sync_copy(merged_v.at[pl.ds(0, _L)], sel_hbm.at[pl.ds(img * _L, _L)])


@functools.partial(
    pl.kernel,
    mesh=plsc.VectorSubcoreMesh(core_axis_name="c", subcore_axis_name="s"),
    out_type=jax.ShapeDtypeStruct((_NIMG * _L,), jnp.int32),
    compiler_params=pltpu.CompilerParams(needs_layout_passes=False),
    scratch_types=[
        pltpu.VMEM((_CHUNK,), jnp.int32),
        pltpu.VMEM((_L * _HROW,), jnp.int32),
        pltpu.VMEM((_HROW,), jnp.int32),
        pltpu.VMEM((_WIN,), jnp.float32),
        pltpu.VMEM((_WIN,), jnp.int32),
        pltpu.VMEM((_WIN,), jnp.float32),
        pltpu.VMEM((_WIN,), jnp.int32),
        pltpu.SemaphoreType.DMA,
        pltpu.SemaphoreType.DMA,
        pltpu.VMEM_SHARED((_L, _HROW), jnp.int32),
    ],
)
def _select(
    pred_hbm, mask_hbm, sel_hbm, keys_v, hist_v, merged_v,
    winp0, winm0, winp1, winm1, sem0, sem1, shared,
):
    _sc_select(
        pred_hbm, mask_hbm, sel_hbm, keys_v, hist_v, merged_v,
        winp0, winm0, winp1, winm1, sem0, sem1, shared,
    )


@jax.jit
def kernel(pred, mask):
    if pred.ndim == 4 and pred.shape[1] == 1:
        pred = pred[:, 0]
    b = pred.shape[0]
    pred3 = pred.reshape(b, _ROWS, _COLS)
    mask3 = mask.astype(jnp.int32).reshape(b, _ROWS, _COLS)

    sel = _select(
        pred.reshape(b * _PARTS, _CHUNK), mask3.reshape(b * _PARTS, _CHUNK)
    )

    key, bce, cls = pl.pallas_call(
        _pass_p,
        grid=(b,),
        in_specs=[
            pl.BlockSpec((1, _ROWS, _COLS), lambda i: (i, 0, 0)),
            pl.BlockSpec((1, _ROWS, _COLS), lambda i: (i, 0, 0)),
        ],
        out_specs=[
            pl.BlockSpec((1, _ROWS, _COLS), lambda i: (i, 0, 0)),
            pl.BlockSpec((1, _ROWS, _COLS), lambda i: (i, 0, 0)),
            pl.BlockSpec((1, _ROWS, _COLS), lambda i: (i, 0, 0)),
        ],
        out_shape=[
            jax.ShapeDtypeStruct((b, _ROWS, _COLS), jnp.int32),
            jax.ShapeDtypeStruct((b, _ROWS, _COLS), jnp.float32),
            jax.ShapeDtypeStruct((b, _ROWS, _COLS), jnp.int8),
        ],
    )(pred3, mask3)

    train8, hold8, parts = pl.pallas_call(
        _pass_b,
        grid=(b,),
        in_specs=[
            pl.BlockSpec((1, _ROWS, _COLS), lambda i: (i, 0, 0)),
            pl.BlockSpec((1, _ROWS, _COLS), lambda i: (i, 0, 0)),
            pl.BlockSpec((1, _ROWS, _COLS), lambda i: (i, 0, 0)),
            pl.BlockSpec((1, 1, _L), lambda i: (i, 0, 0), memory_space=pltpu.SMEM),
        ],
        out_specs=[
            pl.BlockSpec((1, _ROWS, _COLS), lambda i: (i, 0, 0)),
            pl.BlockSpec((1, _ROWS, _COLS), lambda i: (i, 0, 0)),
            pl.BlockSpec((1, 1, _COLS), lambda i: (0, 0, 0)),
        ],
        out_shape=[
            jax.ShapeDtypeStruct((b, _ROWS, _COLS), jnp.bool_),
            jax.ShapeDtypeStruct((b, _ROWS, _COLS), jnp.bool_),
            jax.ShapeDtypeStruct((1, 1, _COLS), jnp.float32),
        ],
    )(key, bce, cls, sel.reshape(b, 1, _L))

    train = train8.reshape(b, 512, 512)
    hold = hold8.reshape(b, 512, 512)
    fin = parts[0, 0]
    loss = fin[0]
    stats = fin[1:12]
    return loss, train, hold, stats
